# Initial kernel scaffold; baseline (speedup 1.0000x reference)
import jax
import jax.numpy as jnp
from jax.experimental import pallas as pl


def kernel(feature, edge_index, W1, b1, W2, b2):
    raise NotImplementedError("write your pallas kernel here")


# speedup vs baseline: 33.6726x; 33.6726x over previous
import functools

import jax
import jax.numpy as jnp
from jax import lax
from jax.experimental import pallas as pl
from jax.experimental.pallas import tpu as pltpu
from jax.experimental.pallas import tpu_sc as plsc

N = 10000
D = 128
H = 32
C = 40
E = 320000

NP = 10240
NC = 2
NS = 16
NW = NC * NS
B = 128
NB = (E + NW * B - 1) // (NW * B)
EP = NW * NB * B
ROWS = NP // NS


def _sc_deg_body(dst_hbm, ones_hbm, zeros_hbm, out_hbm, dst_v, ones_v, deg_sh):
    c = lax.axis_index("c")
    s = lax.axis_index("s")
    wid = c * NS + s
    pltpu.sync_copy(zeros_hbm.at[pl.ds(s * ROWS, ROWS)],
                    deg_sh.at[pl.ds(s * ROWS, ROWS)])
    pltpu.sync_copy(dst_hbm.at[wid], dst_v)
    pltpu.sync_copy(ones_hbm, ones_v)
    plsc.subcore_barrier()

    def body(j, carry):
        pltpu.sync_copy(ones_v, deg_sh.at[dst_v.at[j]], add=True)
        return carry

    lax.fori_loop(0, NB, body, 0)
    plsc.subcore_barrier()
    pltpu.sync_copy(deg_sh.at[pl.ds(s * ROWS, ROWS)],
                    out_hbm.at[c, pl.ds(s * ROWS, ROWS)])


def _sc_msg_body(g_hbm, src_hbm, dst_hbm, zeros_hbm, out_hbm,
                 src_v, dst_v, rows_v, acc_sh, gsem):
    c = lax.axis_index("c")
    s = lax.axis_index("s")
    wid = c * NS + s
    pltpu.sync_copy(zeros_hbm.at[pl.ds(s * ROWS, ROWS)],
                    acc_sh.at[pl.ds(s * ROWS, ROWS)])
    pltpu.sync_copy(src_hbm.at[wid], src_v)
    pltpu.sync_copy(dst_hbm.at[wid], dst_v)
    plsc.subcore_barrier()

    def body(j, carry):
        pltpu.async_copy(g_hbm.at[src_v.at[j]], rows_v, gsem).wait()
        pltpu.sync_copy(rows_v, acc_sh.at[dst_v.at[j]], add=True)
        return carry

    lax.fori_loop(0, NB, body, 0)
    plsc.subcore_barrier()
    pltpu.sync_copy(acc_sh.at[pl.ds(s * ROWS, ROWS)],
                    out_hbm.at[c, pl.ds(s * ROWS, ROWS)])


def _tc_a_body(feat_ref, w1_ref, degp_ref, g_ref, dinv_ref):
    deg = degp_ref[:, 0:1] + degp_ref[:, 1:2] + 1.0
    dinv = lax.rsqrt(deg)
    h = jnp.dot(feat_ref[...], w1_ref[...], preferred_element_type=jnp.float32)
    g_ref[...] = h * dinv
    dinv_ref[...] = dinv


def _tc_b_body(g_ref, s0_ref, s1_ref, dinv_ref, b1_ref, w2_ref, b2_ref, out_ref):
    t = (s0_ref[...] + s1_ref[...] + g_ref[...]) * dinv_ref[...]
    t = jnp.maximum(t + b1_ref[...], 0.0)
    z = jnp.dot(t, w2_ref[...], preferred_element_type=jnp.float32) + b2_ref[...]
    m = jnp.max(z, axis=1, keepdims=True)
    lse = jnp.log(jnp.sum(jnp.exp(z - m), axis=1, keepdims=True)) + m
    out_ref[...] = z - lse


_sc_mesh = plsc.VectorSubcoreMesh(core_axis_name="c", subcore_axis_name="s")
_sc_params = pltpu.CompilerParams(use_tc_tiling_on_sc=False)

_deg_call = pl.kernel(
    _sc_deg_body,
    out_type=jax.ShapeDtypeStruct((NC, NP), jnp.float32),
    mesh=_sc_mesh,
    compiler_params=_sc_params,
    scratch_types=[
        pltpu.VMEM((NB, B), jnp.int32),
        pltpu.VMEM((B,), jnp.float32),
        pltpu.VMEM_SHARED((NP,), jnp.float32),
    ],
)

_msg_call = pl.kernel(
    _sc_msg_body,
    out_type=jax.ShapeDtypeStruct((NC, NP, H), jnp.float32),
    mesh=_sc_mesh,
    compiler_params=_sc_params,
    scratch_types=[
        pltpu.VMEM((NB, B), jnp.int32),
        pltpu.VMEM((NB, B), jnp.int32),
        pltpu.VMEM((B, H), jnp.float32),
        pltpu.VMEM_SHARED((NP, H), jnp.float32),
        pltpu.SemaphoreType.DMA,
    ],
)

_tc_a_call = pl.pallas_call(
    _tc_a_body,
    out_shape=[
        jax.ShapeDtypeStruct((NP, H), jnp.float32),
        jax.ShapeDtypeStruct((NP, 1), jnp.float32),
    ],
)

_tc_b_call = pl.pallas_call(
    _tc_b_body,
    out_shape=jax.ShapeDtypeStruct((NP, C), jnp.float32),
)


def kernel(feature, edge_index, W1, b1, W2, b2):
    ei = edge_index.astype(jnp.int32)
    pad = jnp.full((EP - E,), N, dtype=jnp.int32)
    src = jnp.concatenate([ei[0], pad]).reshape(NW, NB, B)
    dst = jnp.concatenate([ei[1], pad]).reshape(NW, NB, B)

    feat_p = jnp.concatenate(
        [feature, jnp.zeros((NP - N, D), jnp.float32)], axis=0)
    ones_b = jnp.ones((B,), jnp.float32)
    zeros_n = jnp.zeros((NP,), jnp.float32)
    zeros_nh = jnp.zeros((NP, H), jnp.float32)

    degp = _deg_call(dst, ones_b, zeros_n)
    degp_t = jnp.transpose(degp)

    g, dinv = _tc_a_call(feat_p, W1, degp_t)

    partials = _msg_call(g, src, dst, zeros_nh)

    out = _tc_b_call(g, partials[0], partials[1], dinv,
                     b1.reshape(1, H), W2, b2.reshape(1, C))
    return out[:N]


# speedup vs baseline: 34.3692x; 1.0207x over previous
import functools

import jax
import jax.numpy as jnp
from jax import lax
from jax.experimental import pallas as pl
from jax.experimental.pallas import tpu as pltpu
from jax.experimental.pallas import tpu_sc as plsc

N = 10000
D = 128
H = 32
C = 40
E = 320000

NP = 10240
NC = 2
NS = 16
NW = NC * NS
B = 128
NB = 80
EP = NW * NB * B
ROWS = NP // NS


def _sc_deg_body(dst_hbm, ones_hbm, zeros_hbm, out_hbm, dst_v, ones_v, deg_sh,
                 dsem):
    c = lax.axis_index("c")
    s = lax.axis_index("s")
    wid = c * NS + s
    pltpu.sync_copy(zeros_hbm.at[pl.ds(s * ROWS, ROWS)],
                    deg_sh.at[pl.ds(s * ROWS, ROWS)])
    pltpu.sync_copy(dst_hbm.at[wid], dst_v)
    pltpu.sync_copy(ones_hbm, ones_v)
    plsc.subcore_barrier()

    def fire(j, carry):
        pltpu.async_copy(ones_v.at[j], deg_sh.at[dst_v.at[j]], dsem, add=True)
        return carry

    lax.fori_loop(0, NB, fire, 0)

    def drain(j, carry):
        pltpu.make_async_copy(ones_v.at[j], deg_sh.at[dst_v.at[j]], dsem).wait()
        return carry

    lax.fori_loop(0, NB, drain, 0)
    plsc.subcore_barrier()
    pltpu.sync_copy(deg_sh.at[pl.ds(s * ROWS, ROWS)],
                    out_hbm.at[c, pl.ds(s * ROWS, ROWS)])


def _sc_msg_body(g_hbm, src_hbm, dst_hbm, zeros_hbm, out_hbm,
                 src_v, dst_v, buf0, buf1, acc_sh, gsem0, gsem1):
    c = lax.axis_index("c")
    s = lax.axis_index("s")
    wid = c * NS + s
    pltpu.sync_copy(zeros_hbm.at[pl.ds(s * ROWS, ROWS)],
                    acc_sh.at[pl.ds(s * ROWS, ROWS)])
    pltpu.sync_copy(src_hbm.at[wid], src_v)
    pltpu.sync_copy(dst_hbm.at[wid], dst_v)
    plsc.subcore_barrier()

    def gather(j, buf, sem):
        pltpu.async_copy(g_hbm.at[src_v.at[j]], buf, sem)

    def gather_wait(j, buf, sem):
        pltpu.make_async_copy(g_hbm.at[src_v.at[j]], buf, sem).wait()

    def scatter(j, buf):
        pltpu.sync_copy(buf, acc_sh.at[dst_v.at[j]], add=True)

    gather(0, buf0, gsem0)
    gather(1, buf1, gsem1)

    def body(k, carry):
        j0 = 2 * k
        j1 = 2 * k + 1
        gather_wait(j0, buf0, gsem0)
        scatter(j0, buf0)
        gather(lax.rem(j0 + 2, NB), buf0, gsem0)
        gather_wait(j1, buf1, gsem1)
        scatter(j1, buf1)
        gather(lax.rem(j1 + 2, NB), buf1, gsem1)
        return carry

    lax.fori_loop(0, NB // 2, body, 0)
    gather_wait(0, buf0, gsem0)
    gather_wait(1, buf1, gsem1)
    plsc.subcore_barrier()
    pltpu.sync_copy(acc_sh.at[pl.ds(s * ROWS, ROWS)],
                    out_hbm.at[c, pl.ds(s * ROWS, ROWS)])


def _tc_a_body(feat_ref, w1_ref, degp_ref, g_ref, dinv_ref):
    deg = degp_ref[:, 0:1] + degp_ref[:, 1:2] + 1.0
    dinv = lax.rsqrt(deg)
    h = jnp.dot(feat_ref[...], w1_ref[...], preferred_element_type=jnp.float32)
    g_ref[...] = h * dinv
    dinv_ref[...] = dinv


def _tc_b_body(g_ref, s0_ref, s1_ref, dinv_ref, b1_ref, w2_ref, b2_ref, out_ref):
    t = (s0_ref[...] + s1_ref[...] + g_ref[...]) * dinv_ref[...]
    t = jnp.maximum(t + b1_ref[...], 0.0)
    z = jnp.dot(t, w2_ref[...], preferred_element_type=jnp.float32) + b2_ref[...]
    m = jnp.max(z, axis=1, keepdims=True)
    lse = jnp.log(jnp.sum(jnp.exp(z - m), axis=1, keepdims=True)) + m
    out_ref[...] = z - lse


_sc_mesh = plsc.VectorSubcoreMesh(core_axis_name="c", subcore_axis_name="s")
_sc_params = pltpu.CompilerParams(use_tc_tiling_on_sc=False)

_deg_call = pl.kernel(
    _sc_deg_body,
    out_type=jax.ShapeDtypeStruct((NC, NP), jnp.float32),
    mesh=_sc_mesh,
    compiler_params=_sc_params,
    scratch_types=[
        pltpu.VMEM((NB, B), jnp.int32),
        pltpu.VMEM((NB, B), jnp.float32),
        pltpu.VMEM_SHARED((NP,), jnp.float32),
        pltpu.SemaphoreType.DMA,
    ],
)

_msg_call = pl.kernel(
    _sc_msg_body,
    out_type=jax.ShapeDtypeStruct((NC, NP, H), jnp.float32),
    mesh=_sc_mesh,
    compiler_params=_sc_params,
    scratch_types=[
        pltpu.VMEM((NB, B), jnp.int32),
        pltpu.VMEM((NB, B), jnp.int32),
        pltpu.VMEM((B, H), jnp.float32),
        pltpu.VMEM((B, H), jnp.float32),
        pltpu.VMEM_SHARED((NP, H), jnp.float32),
        pltpu.SemaphoreType.DMA,
        pltpu.SemaphoreType.DMA,
    ],
)

_tc_a_call = pl.pallas_call(
    _tc_a_body,
    out_shape=[
        jax.ShapeDtypeStruct((NP, H), jnp.float32),
        jax.ShapeDtypeStruct((NP, 1), jnp.float32),
    ],
)

_tc_b_call = pl.pallas_call(
    _tc_b_body,
    out_shape=jax.ShapeDtypeStruct((NP, C), jnp.float32),
)


def kernel(feature, edge_index, W1, b1, W2, b2):
    ei = edge_index.astype(jnp.int32)
    pad = jnp.full((EP - E,), N, dtype=jnp.int32)
    src = jnp.concatenate([ei[0], pad]).reshape(NW, NB, B)
    dst = jnp.concatenate([ei[1], pad]).reshape(NW, NB, B)

    feat_p = jnp.concatenate(
        [feature, jnp.zeros((NP - N, D), jnp.float32)], axis=0)
    ones_b = jnp.ones((NB, B), jnp.float32)
    zeros_n = jnp.zeros((NP,), jnp.float32)
    zeros_nh = jnp.zeros((NP, H), jnp.float32)

    degp = _deg_call(dst, ones_b, zeros_n)
    degp_t = jnp.transpose(degp)

    g, dinv = _tc_a_call(feat_p, W1, degp_t)

    partials = _msg_call(g, src, dst, zeros_nh)

    out = _tc_b_call(g, partials[0], partials[1], dinv,
                     b1.reshape(1, H), W2, b2.reshape(1, C))
    return out[:N]


# speedup vs baseline: 39.9584x; 1.1626x over previous
import jax
import jax.numpy as jnp
from jax import lax
from jax.experimental import pallas as pl
from jax.experimental.pallas import tpu as pltpu
from jax.experimental.pallas import tpu_sc as plsc

N = 10000
D = 128
H = 32
C = 40
E = 320000

NP = 10240
NC = 2
NS = 16
NW = NC * NS
B = 128
NB = 80
EP = NW * NB * B
ROWS = NP // NS


def _sc_deg_body(dst_hbm, ones_hbm, zeros_hbm, out_hbm, dst_v, ones_v, deg_sh,
                 dsem):
    c = lax.axis_index("c")
    s = lax.axis_index("s")
    wid = c * NS + s
    pltpu.sync_copy(zeros_hbm.at[pl.ds(s * ROWS, ROWS)],
                    deg_sh.at[pl.ds(s * ROWS, ROWS)])
    pltpu.sync_copy(dst_hbm.at[wid], dst_v)
    pltpu.sync_copy(ones_hbm, ones_v)
    plsc.subcore_barrier()

    def fire(j, carry):
        pltpu.async_copy(ones_v.at[j], deg_sh.at[dst_v.at[j]], dsem, add=True)
        return carry

    lax.fori_loop(0, NB, fire, 0)

    def drain(j, carry):
        pltpu.make_async_copy(ones_v.at[j], deg_sh.at[dst_v.at[j]], dsem).wait()
        return carry

    lax.fori_loop(0, NB, drain, 0)
    plsc.subcore_barrier()
    pltpu.sync_copy(deg_sh.at[pl.ds(s * ROWS, ROWS)],
                    out_hbm.at[c, pl.ds(s * ROWS, ROWS)])


def _sc_msg_body(g_hbm, src_hbm, dst_hbm, zeros_hbm, out_hbm, src_v, dst_v,
                 b0, b1, b2, b3, acc_sh,
                 g0, g1, g2, g3, s0, s1, s2, s3):
    c = lax.axis_index("c")
    s = lax.axis_index("s")
    wid = c * NS + s
    bufs = (b0, b1, b2, b3)
    gsems = (g0, g1, g2, g3)
    ssems = (s0, s1, s2, s3)
    pltpu.sync_copy(zeros_hbm.at[pl.ds(s * ROWS, ROWS)],
                    acc_sh.at[pl.ds(s * ROWS, ROWS)])
    pltpu.sync_copy(src_hbm.at[wid], src_v)
    pltpu.sync_copy(dst_hbm.at[wid], dst_v)
    plsc.subcore_barrier()

    def gather(j, i):
        pltpu.async_copy(g_hbm.at[src_v.at[j]], bufs[i], gsems[i])

    def gather_wait(j, i):
        pltpu.make_async_copy(g_hbm.at[src_v.at[j]], bufs[i], gsems[i]).wait()

    def scatter(j, i):
        pltpu.async_copy(bufs[i], acc_sh.at[dst_v.at[j]], ssems[i], add=True)

    def scatter_wait(j, i):
        pltpu.make_async_copy(
            bufs[i], acc_sh.at[dst_v.at[j]], ssems[i]).wait()

    gather(0, 0)
    gather(1, 1)

    def body(k, carry):
        for i in range(4):
            j = 4 * k + i
            gather_wait(j, i)
            scatter(j, i)
            nxt = (i + 2) % 4

            @pl.when(j >= 2)
            def _():
                scatter_wait(j - 2, nxt)

            gather(lax.rem(j + 2, NB), nxt)
        return carry

    lax.fori_loop(0, NB // 4, body, 0)
    gather_wait(0, 0)
    gather_wait(1, 1)
    scatter_wait(NB - 2, 2)
    scatter_wait(NB - 1, 3)
    plsc.subcore_barrier()
    pltpu.sync_copy(acc_sh.at[pl.ds(s * ROWS, ROWS)],
                    out_hbm.at[c, pl.ds(s * ROWS, ROWS)])


def _tc_h_body(feat_ref, w1_ref, h_ref):
    hh = jnp.dot(feat_ref[...], w1_ref[...], preferred_element_type=jnp.float32)
    h_ref[...] = jnp.concatenate(
        [hh, jnp.zeros((NP - N, H), jnp.float32)], axis=0)


def _tc_g_body(h_ref, degp_ref, g_ref, dinv_ref):
    deg = degp_ref[0, :] + degp_ref[1, :] + 1.0
    dinv = lax.rsqrt(deg).reshape(NP, 1)
    g_ref[...] = h_ref[...] * dinv
    dinv_ref[...] = dinv


def _tc_head_body(g_ref, p_ref, dinv_ref, b1_ref, w2_ref, b2_ref, out_ref):
    t = (p_ref[0] + p_ref[1] + g_ref[...]) * dinv_ref[...]
    t = jnp.maximum(t + b1_ref[...], 0.0)
    z = jnp.dot(t, w2_ref[...], preferred_element_type=jnp.float32) + b2_ref[...]
    m = jnp.max(z, axis=1, keepdims=True)
    lse = jnp.log(jnp.sum(jnp.exp(z - m), axis=1, keepdims=True)) + m
    out_ref[...] = (z - lse)[:N]


_sc_mesh = plsc.VectorSubcoreMesh(core_axis_name="c", subcore_axis_name="s")
_sc_params = pltpu.CompilerParams(use_tc_tiling_on_sc=False)

_deg_call = pl.kernel(
    _sc_deg_body,
    out_type=jax.ShapeDtypeStruct((NC, NP), jnp.float32),
    mesh=_sc_mesh,
    compiler_params=_sc_params,
    scratch_types=[
        pltpu.VMEM((NB, B), jnp.int32),
        pltpu.VMEM((NB, B), jnp.float32),
        pltpu.VMEM_SHARED((NP,), jnp.float32),
        pltpu.SemaphoreType.DMA,
    ],
)

_msg_call = pl.kernel(
    _sc_msg_body,
    out_type=jax.ShapeDtypeStruct((NC, NP, H), jnp.float32),
    mesh=_sc_mesh,
    compiler_params=_sc_params,
    scratch_types=(
        [pltpu.VMEM((NB, B), jnp.int32)] * 2
        + [pltpu.VMEM((B, H), jnp.float32)] * 4
        + [pltpu.VMEM_SHARED((NP, H), jnp.float32)]
        + [pltpu.SemaphoreType.DMA] * 8
    ),
)

_tc_h_call = pl.pallas_call(
    _tc_h_body,
    out_shape=jax.ShapeDtypeStruct((NP, H), jnp.float32),
)

_tc_g_call = pl.pallas_call(
    _tc_g_body,
    out_shape=[
        jax.ShapeDtypeStruct((NP, H), jnp.float32),
        jax.ShapeDtypeStruct((NP, 1), jnp.float32),
    ],
)

_tc_head_call = pl.pallas_call(
    _tc_head_body,
    out_shape=jax.ShapeDtypeStruct((N, C), jnp.float32),
)


def kernel(feature, edge_index, W1, b1, W2, b2):
    ei = edge_index.astype(jnp.int32)
    pad = jnp.full((EP - E,), N, dtype=jnp.int32)
    src = jnp.concatenate([ei[0], pad]).reshape(NW, NB, B)
    dst = jnp.concatenate([ei[1], pad]).reshape(NW, NB, B)

    ones_b = jnp.ones((NB, B), jnp.float32)
    zeros_n = jnp.zeros((NP,), jnp.float32)
    zeros_nh = jnp.zeros((NP, H), jnp.float32)

    degp = _deg_call(dst, ones_b, zeros_n)
    h = _tc_h_call(feature, W1)

    g, dinv = _tc_g_call(h, degp)

    partials = _msg_call(g, src, dst, zeros_nh)

    return _tc_head_call(g, partials, dinv, b1, W2, b2)
